# baseline (speedup 1.0000x reference)
import functools

import jax
import jax.numpy as jnp
from jax import lax
from jax.experimental import pallas as pl
from jax.experimental.pallas import tpu as pltpu
from jax.experimental.pallas import tpu_sc as plsc

NV, D, NN, B = 2, 16, 1000000, 4096
NH, NR, NG = 5, 3, 10
NW = 32
WPV = NW // NV
PPW = B // WPV
CE = 256
_REG_NODES, _REG_NEIGH, _REG_ROLE0, _REG_ROLE1 = 0, 1, 6, 9
_REG_NEGM, _REG_NEG2, _REG_NEG3, _REG_NEG4A, _REG_NEG4B = 12, 62, 72, 122, 152
N_CHUNKS = 182
IDXW = N_CHUNKS * CE

_mesh = plsc.VectorSubcoreMesh(
    core_axis_name="c", subcore_axis_name="s", num_cores=2, num_subcores=16)


def _splat(v):
    return jnp.full((16,), v, jnp.int32)


def _log_sigmoid(x):
    u = jnp.exp(-jnp.abs(x))
    t = 1.0 + u
    bits = lax.bitcast_convert_type(t, jnp.int32)
    ex = ((bits >> 23) - 127).astype(jnp.float32)
    m = lax.bitcast_convert_type((bits & 0x007FFFFF) | 0x3F800000, jnp.float32)
    s = (m - 1.0) / (m + 1.0)
    s2 = s * s
    p = jnp.float32(1.0 / 9.0)
    p = jnp.float32(1.0 / 7.0) + s2 * p
    p = jnp.float32(1.0 / 5.0) + s2 * p
    p = jnp.float32(1.0 / 3.0) + s2 * p
    p = jnp.float32(1.0) + s2 * p
    l1p = ex * jnp.float32(0.6931471805599453) + 2.0 * s * p
    return jnp.minimum(x, 0.0) - l1p


@functools.partial(
    pl.kernel,
    out_type=jax.ShapeDtypeStruct((NW, 16), jnp.float32),
    mesh=_mesh,
    compiler_params=pltpu.CompilerParams(
        use_tc_tiling_on_sc=False, needs_layout_passes=False),
    scratch_types=[
        pltpu.VMEM((IDXW,), jnp.int32),
        pltpu.VMEM((PPW, 16), jnp.float32),
        pltpu.VMEM((4, 2 * CE, 16), jnp.float32),
        pltpu.VMEM((16,), jnp.float32),
        pltpu.VMEM((16,), jnp.float32),
        pltpu.SemaphoreType.DMA,
        pltpu.SemaphoreType.DMA,
        pltpu.SemaphoreType.DMA((4,)),
    ],
)
def _sc_loss(node_t, neigh_t, nodes_ib, neighs_ib, roles_ib,
             negm, neg2, neg3, neg4, wvec_hbm, out_hbm,
             idxflat, qbuf, rows, wbuf, accv, sems, semq, semr):
    wid = lax.axis_index("s") * 2 + lax.axis_index("c")
    pltpu.sync_copy(wvec_hbm, wbuf)
    iota16 = lax.iota(jnp.int32, 16)

    def compute_chunk(c, par, ce, inv_k, sign, wsplat, acc):
        psplat = _splat(par)
        ebase0 = c * ce

        def gbody(g, acc2):
            ridx = _splat(g * 16) + iota16
            e_f = (_splat(g * 16) + iota16).astype(jnp.float32) + (
                ebase0.astype(jnp.float32))
            bvec = ((e_f + 0.5) * inv_k).astype(jnp.int32)
            dot = (plsc.load_gather(rows, [psplat, ridx, _splat(0)])
                   * plsc.load_gather(qbuf, [bvec, _splat(0)]))
            for d in range(1, 16):
                dot = dot + (plsc.load_gather(rows, [psplat, ridx, _splat(d)])
                             * plsc.load_gather(qbuf, [bvec, _splat(d)]))
            x = dot if sign > 0 else -dot
            return acc2 + wsplat * _log_sigmoid(x)

        return lax.fori_loop(0, ce // 16, gbody, acc)

    def run_segment(tab, ebase, n, ce, inv_k, sign, wslot, acc):
        wsplat = plsc.load_gather(wbuf, [_splat(wslot)])

        def issue(c, par):
            pltpu.async_copy(
                tab.at[idxflat.at[pl.ds(ebase + c * ce, ce)]],
                rows.at[par, pl.ds(0, ce)], semr.at[par])

        def drain(par):
            pltpu.make_async_copy(
                node_t.at[0, pl.ds(0, ce)],
                rows.at[par, pl.ds(0, ce)], semr.at[par]).wait()

        for p in range(min(3, n)):
            issue(p, p)

        def cbody(c, acc2):
            par = lax.rem(c, 4)

            @pl.when(c + 3 < n)
            def _():
                issue(c + 3, lax.rem(c + 3, 4))

            drain(par)
            return compute_chunk(c, par, ce, inv_k, sign, wsplat, acc2)

        return lax.fori_loop(0, n, cbody, acc)

    def block(i):
        j = 1 - i
        wl = wid - i * WPV
        b0 = wl * PPW
        regions = [
            (_REG_NODES, 1, nodes_ib.at[i, pl.ds(b0, PPW)]),
            (_REG_NEIGH, 5, neighs_ib.at[i, pl.ds(b0 * NH, PPW * NH)]),
            (_REG_ROLE0, 3, roles_ib.at[i, 0, pl.ds(b0 * NR, PPW * NR)]),
            (_REG_ROLE1, 3, roles_ib.at[i, 1, pl.ds(b0 * NR, PPW * NR)]),
            (_REG_NEGM, 50, negm.at[i, pl.ds(b0 * NH * NG, PPW * NH * NG)]),
            (_REG_NEG2, 10, neg2.at[i, j, pl.ds(b0 * NG, PPW * NG)]),
            (_REG_NEG3, 50, neg3.at[i, j, pl.ds(b0 * NH * NG, PPW * NH * NG)]),
            (_REG_NEG4A, 30, neg4.at[i, 0, pl.ds(b0 * NR * NG, PPW * NR * NG)]),
            (_REG_NEG4B, 30, neg4.at[i, 1, pl.ds(b0 * NR * NG, PPW * NR * NG)]),
        ]
        for base, n, src in regions:
            pltpu.async_copy(src, idxflat.at[pl.ds(base * CE, n * CE)], sems)
        for base, n, src in regions:
            pltpu.make_async_copy(
                negm.at[0, pl.ds(0, n * CE)],
                idxflat.at[pl.ds(base * CE, n * CE)], sems).wait()
        for q in range(2):
            pltpu.async_copy(
                node_t.at[i].at[idxflat.at[pl.ds(_REG_NODES * CE + q * 128, 128)]],
                qbuf.at[pl.ds(q * 128, 128)], semq)
        for q in range(2):
            pltpu.make_async_copy(node_t.at[0, pl.ds(0, 128)],
                                  qbuf.at[pl.ds(q * 128, 128)], semq).wait()

        inv5 = jnp.float32(1.0 / NH)
        inv50 = jnp.float32(1.0 / (NH * NG))
        inv10 = jnp.float32(1.0 / NG)
        inv3 = jnp.float32(1.0 / NR)
        inv30 = jnp.float32(1.0 / (NR * NG))
        one = jnp.float32(1.0)
        segs = [
            (neigh_t.at[i], _REG_NEIGH, 5, 256, inv5, 1, 0),
            (neigh_t.at[i], _REG_NEGM, 25, 512, inv50, -1, 1),
            (node_t.at[j], _REG_NODES, 1, 256, one, 1, 2),
            (node_t.at[j], _REG_NEG2, 5, 512, inv10, -1, 2),
            (neigh_t.at[j], _REG_NEIGH, 5, 256, inv5, 1, 3),
            (neigh_t.at[j], _REG_NEG3, 25, 512, inv50, -1, 4),
            (node_t.at[0], _REG_ROLE0, 3, 256, inv3, 1, 5),
            (node_t.at[1], _REG_ROLE1, 3, 256, inv3, 1, 5),
            (node_t.at[0], _REG_NEG4A, 15, 512, inv30, -1, 6),
            (node_t.at[1], _REG_NEG4B, 15, 512, inv30, -1, 6),
        ]
        acc = jnp.zeros((16,), jnp.float32)
        for tab, reg0, n, ce, inv_k, sign, wslot in segs:
            acc = run_segment(tab, reg0 * CE, n, ce, inv_k, sign, wslot, acc)
        accv[...] = acc

    @pl.when(wid < WPV)
    def _():
        block(0)

    @pl.when(wid >= WPV)
    def _():
        block(1)

    pltpu.sync_copy(accv, out_hbm.at[wid])


def kernel(node_emb_tables, neigh_emb_tables, hyp1, hyp2, hyp3, count,
           shuffle_indices_nets, nodes_idx_nets, neigh_idx_nets,
           node_role_nets, neg_main, neg2, neg3, neg4):
    bidx = [lax.dynamic_slice_in_dim(shuffle_indices_nets[i], count, B)
            for i in range(NV)]
    nodes_ib = jnp.stack([nodes_idx_nets[i][bidx[i]] for i in range(NV)])
    neighs_ib = jnp.stack(
        [neigh_idx_nets[i][bidx[i]].reshape(-1) for i in range(NV)])
    roles_ib = jnp.stack([
        jnp.stack([node_role_nets[i, jj][bidx[i]].reshape(-1)
                   for jj in range(NV)]) for i in range(NV)])

    Bf = jnp.float32(B)
    scale = jnp.float32(-0.1)
    wvec = jnp.stack([
        1.0 / (NH * Bf), 1.0 / Bf, hyp1 / Bf, hyp2 / (NH * Bf), hyp2 / Bf,
        hyp3 / (NR * Bf), hyp3 / Bf,
        0.0, 0.0, 0.0, 0.0, 0.0, 0.0, 0.0, 0.0, 0.0,
    ]).astype(jnp.float32) * scale

    out = _sc_loss(node_emb_tables, neigh_emb_tables,
                   nodes_ib.astype(jnp.int32), neighs_ib.astype(jnp.int32),
                   roles_ib.astype(jnp.int32), neg_main, neg2, neg3, neg4,
                   wvec)
    return jnp.sum(out)


# speedup vs baseline: 1.0567x; 1.0567x over previous
import functools

import jax
import jax.numpy as jnp
from jax import lax
from jax.experimental import pallas as pl
from jax.experimental.pallas import tpu as pltpu
from jax.experimental.pallas import tpu_sc as plsc

NV, D, NN, B = 2, 16, 1000000, 4096
NH, NR, NG = 5, 3, 10
NW = 32
WPV = NW // NV
PPW = B // WPV
CE = 256
_REG_NODES, _REG_NEIGH, _REG_ROLE0, _REG_ROLE1 = 0, 1, 6, 9
_REG_NEGM, _REG_NEG2, _REG_NEG3, _REG_NEG4A, _REG_NEG4B = 12, 62, 72, 122, 152
N_CHUNKS = 182
IDXW = N_CHUNKS * CE

_mesh = plsc.VectorSubcoreMesh(
    core_axis_name="c", subcore_axis_name="s", num_cores=2, num_subcores=16)


def _splat(v):
    return jnp.full((16,), v, jnp.int32)


def _log_sigmoid(x):
    u = jnp.exp(-jnp.abs(x))
    t = 1.0 + u
    bits = lax.bitcast_convert_type(t, jnp.int32)
    ex = ((bits >> 23) - 127).astype(jnp.float32)
    m = lax.bitcast_convert_type((bits & 0x007FFFFF) | 0x3F800000, jnp.float32)
    s = (m - 1.0) / (m + 1.0)
    s2 = s * s
    p = jnp.float32(1.0 / 9.0)
    p = jnp.float32(1.0 / 7.0) + s2 * p
    p = jnp.float32(1.0 / 5.0) + s2 * p
    p = jnp.float32(1.0 / 3.0) + s2 * p
    p = jnp.float32(1.0) + s2 * p
    l1p = ex * jnp.float32(0.6931471805599453) + 2.0 * s * p
    return jnp.minimum(x, 0.0) - l1p


@functools.partial(
    pl.kernel,
    out_type=jax.ShapeDtypeStruct((NW, 16), jnp.float32),
    mesh=_mesh,
    compiler_params=pltpu.CompilerParams(
        use_tc_tiling_on_sc=False, needs_layout_passes=False),
    scratch_types=[
        pltpu.VMEM((IDXW,), jnp.int32),
        pltpu.VMEM((PPW, 16), jnp.float32),
        pltpu.VMEM((4, 2 * CE, 16), jnp.float32),
        pltpu.VMEM((16,), jnp.float32),
        pltpu.VMEM((16,), jnp.float32),
        pltpu.SemaphoreType.DMA,
        pltpu.SemaphoreType.DMA,
        pltpu.SemaphoreType.DMA((4,)),
    ],
)
def _sc_loss(node_t, neigh_t, nodes_ib, neighs_ib, roles_ib,
             negm, neg2, neg3, neg4, wvec_hbm, out_hbm,
             idxflat, qbuf, rows, wbuf, accv, sems, semq, semr):
    wid = lax.axis_index("s") * 2 + lax.axis_index("c")
    pltpu.sync_copy(wvec_hbm, wbuf)
    iota16 = lax.iota(jnp.int32, 16)

    def compute_chunk(c, par, ce, inv_k, sign, wsplat, acc):
        psplat = _splat(par)
        ebase0 = c * ce

        def gbody(g, acc2):
            ridx = _splat(g * 16) + iota16
            e_f = (_splat(g * 16) + iota16).astype(jnp.float32) + (
                ebase0.astype(jnp.float32))
            bvec = ((e_f + 0.5) * inv_k).astype(jnp.int32)
            dvec0 = jnp.bitwise_and(iota16, 15)
            dot = (plsc.load_gather(rows, [psplat, ridx, dvec0])
                   * plsc.load_gather(qbuf, [bvec, dvec0]))
            for d in range(1, 16):
                dvec = jnp.bitwise_and(iota16 + d, 15)
                dot = dot + (plsc.load_gather(rows, [psplat, ridx, dvec])
                             * plsc.load_gather(qbuf, [bvec, dvec]))
            x = dot if sign > 0 else -dot
            return acc2 + wsplat * _log_sigmoid(x)

        return lax.fori_loop(0, ce // 16, gbody, acc)

    def run_segment(tab, ebase, n, ce, inv_k, sign, wslot, acc):
        wsplat = plsc.load_gather(wbuf, [_splat(wslot)])

        def issue(c, par):
            pltpu.async_copy(
                tab.at[idxflat.at[pl.ds(ebase + c * ce, ce)]],
                rows.at[par, pl.ds(0, ce)], semr.at[par])

        def drain(par):
            pltpu.make_async_copy(
                node_t.at[0, pl.ds(0, ce)],
                rows.at[par, pl.ds(0, ce)], semr.at[par]).wait()

        for p in range(min(3, n)):
            issue(p, p)

        def cbody(c, acc2):
            par = lax.rem(c, 4)

            @pl.when(c + 3 < n)
            def _():
                issue(c + 3, lax.rem(c + 3, 4))

            drain(par)
            return compute_chunk(c, par, ce, inv_k, sign, wsplat, acc2)

        return lax.fori_loop(0, n, cbody, acc)

    def block(i):
        j = 1 - i
        wl = wid - i * WPV
        b0 = wl * PPW
        regions = [
            (_REG_NODES, 1, nodes_ib.at[i, pl.ds(b0, PPW)]),
            (_REG_NEIGH, 5, neighs_ib.at[i, pl.ds(b0 * NH, PPW * NH)]),
            (_REG_ROLE0, 3, roles_ib.at[i, 0, pl.ds(b0 * NR, PPW * NR)]),
            (_REG_ROLE1, 3, roles_ib.at[i, 1, pl.ds(b0 * NR, PPW * NR)]),
            (_REG_NEGM, 50, negm.at[i, pl.ds(b0 * NH * NG, PPW * NH * NG)]),
            (_REG_NEG2, 10, neg2.at[i, j, pl.ds(b0 * NG, PPW * NG)]),
            (_REG_NEG3, 50, neg3.at[i, j, pl.ds(b0 * NH * NG, PPW * NH * NG)]),
            (_REG_NEG4A, 30, neg4.at[i, 0, pl.ds(b0 * NR * NG, PPW * NR * NG)]),
            (_REG_NEG4B, 30, neg4.at[i, 1, pl.ds(b0 * NR * NG, PPW * NR * NG)]),
        ]
        for base, n, src in regions:
            pltpu.async_copy(src, idxflat.at[pl.ds(base * CE, n * CE)], sems)
        for base, n, src in regions:
            pltpu.make_async_copy(
                negm.at[0, pl.ds(0, n * CE)],
                idxflat.at[pl.ds(base * CE, n * CE)], sems).wait()
        for q in range(2):
            pltpu.async_copy(
                node_t.at[i].at[idxflat.at[pl.ds(_REG_NODES * CE + q * 128, 128)]],
                qbuf.at[pl.ds(q * 128, 128)], semq)
        for q in range(2):
            pltpu.make_async_copy(node_t.at[0, pl.ds(0, 128)],
                                  qbuf.at[pl.ds(q * 128, 128)], semq).wait()

        inv5 = jnp.float32(1.0 / NH)
        inv50 = jnp.float32(1.0 / (NH * NG))
        inv10 = jnp.float32(1.0 / NG)
        inv3 = jnp.float32(1.0 / NR)
        inv30 = jnp.float32(1.0 / (NR * NG))
        one = jnp.float32(1.0)
        segs = [
            (neigh_t.at[i], _REG_NEIGH, 5, 256, inv5, 1, 0),
            (neigh_t.at[i], _REG_NEGM, 25, 512, inv50, -1, 1),
            (node_t.at[j], _REG_NODES, 1, 256, one, 1, 2),
            (node_t.at[j], _REG_NEG2, 5, 512, inv10, -1, 2),
            (neigh_t.at[j], _REG_NEIGH, 5, 256, inv5, 1, 3),
            (neigh_t.at[j], _REG_NEG3, 25, 512, inv50, -1, 4),
            (node_t.at[0], _REG_ROLE0, 3, 256, inv3, 1, 5),
            (node_t.at[1], _REG_ROLE1, 3, 256, inv3, 1, 5),
            (node_t.at[0], _REG_NEG4A, 15, 512, inv30, -1, 6),
            (node_t.at[1], _REG_NEG4B, 15, 512, inv30, -1, 6),
        ]
        acc = jnp.zeros((16,), jnp.float32)
        for tab, reg0, n, ce, inv_k, sign, wslot in segs:
            acc = run_segment(tab, reg0 * CE, n, ce, inv_k, sign, wslot, acc)
        accv[...] = acc

    @pl.when(wid < WPV)
    def _():
        block(0)

    @pl.when(wid >= WPV)
    def _():
        block(1)

    pltpu.sync_copy(accv, out_hbm.at[wid])


def kernel(node_emb_tables, neigh_emb_tables, hyp1, hyp2, hyp3, count,
           shuffle_indices_nets, nodes_idx_nets, neigh_idx_nets,
           node_role_nets, neg_main, neg2, neg3, neg4):
    bidx = [lax.dynamic_slice_in_dim(shuffle_indices_nets[i], count, B)
            for i in range(NV)]
    nodes_ib = jnp.stack([nodes_idx_nets[i][bidx[i]] for i in range(NV)])
    neighs_ib = jnp.stack(
        [neigh_idx_nets[i][bidx[i]].reshape(-1) for i in range(NV)])
    roles_ib = jnp.stack([
        jnp.stack([node_role_nets[i, jj][bidx[i]].reshape(-1)
                   for jj in range(NV)]) for i in range(NV)])

    Bf = jnp.float32(B)
    scale = jnp.float32(-0.1)
    wvec = jnp.stack([
        1.0 / (NH * Bf), 1.0 / Bf, hyp1 / Bf, hyp2 / (NH * Bf), hyp2 / Bf,
        hyp3 / (NR * Bf), hyp3 / Bf,
        0.0, 0.0, 0.0, 0.0, 0.0, 0.0, 0.0, 0.0, 0.0,
    ]).astype(jnp.float32) * scale

    out = _sc_loss(node_emb_tables, neigh_emb_tables,
                   nodes_ib.astype(jnp.int32), neighs_ib.astype(jnp.int32),
                   roles_ib.astype(jnp.int32), neg_main, neg2, neg3, neg4,
                   wvec)
    return jnp.sum(out)


# speedup vs baseline: 1.0659x; 1.0088x over previous
import functools

import jax
import jax.numpy as jnp
from jax import lax
from jax.experimental import pallas as pl
from jax.experimental.pallas import tpu as pltpu
from jax.experimental.pallas import tpu_sc as plsc

NV, D, NN, B = 2, 16, 1000000, 4096
NH, NR, NG = 5, 3, 10
NW = 32
WPV = NW // NV
PPW = B // WPV
CE = 256
_REG_NODES, _REG_NEIGH, _REG_ROLE0, _REG_ROLE1 = 0, 1, 6, 9
_REG_NEGM, _REG_NEG2, _REG_NEG3, _REG_NEG4A, _REG_NEG4B = 12, 62, 72, 122, 152
N_CHUNKS = 182
IDXW = N_CHUNKS * CE

_mesh = plsc.VectorSubcoreMesh(
    core_axis_name="c", subcore_axis_name="s", num_cores=2, num_subcores=16)


def _splat(v):
    return jnp.full((16,), v, jnp.int32)


def _log_sigmoid(x):
    u = jnp.exp(-jnp.abs(x))
    t = 1.0 + u
    bits = lax.bitcast_convert_type(t, jnp.int32)
    ex = ((bits >> 23) - 127).astype(jnp.float32)
    m = lax.bitcast_convert_type((bits & 0x007FFFFF) | 0x3F800000, jnp.float32)
    s = (m - 1.0) / (m + 1.0)
    s2 = s * s
    p = jnp.float32(1.0 / 9.0)
    p = jnp.float32(1.0 / 7.0) + s2 * p
    p = jnp.float32(1.0 / 5.0) + s2 * p
    p = jnp.float32(1.0 / 3.0) + s2 * p
    p = jnp.float32(1.0) + s2 * p
    l1p = ex * jnp.float32(0.6931471805599453) + 2.0 * s * p
    return jnp.minimum(x, 0.0) - l1p


@functools.partial(
    pl.kernel,
    out_type=jax.ShapeDtypeStruct((NW, 16), jnp.float32),
    mesh=_mesh,
    compiler_params=pltpu.CompilerParams(
        use_tc_tiling_on_sc=False, needs_layout_passes=False),
    scratch_types=[
        pltpu.VMEM((IDXW,), jnp.int32),
        pltpu.VMEM((PPW, 16), jnp.float32),
        pltpu.VMEM((4, 2 * CE, 16), jnp.float32),
        pltpu.VMEM((16,), jnp.float32),
        pltpu.VMEM((16,), jnp.float32),
        pltpu.SemaphoreType.DMA,
        pltpu.SemaphoreType.DMA,
        pltpu.SemaphoreType.DMA((4,)),
    ],
)
def _sc_loss(node_t, neigh_t, nodes_ib, neighs_ib, roles_ib,
             negm, neg2, neg3, neg4, wvec_hbm, out_hbm,
             idxflat, qbuf, rows, wbuf, accv, sems, semq, semr):
    wid = lax.axis_index("s") * 2 + lax.axis_index("c")
    pltpu.sync_copy(wvec_hbm, wbuf)
    iota16 = lax.iota(jnp.int32, 16)

    def compute_chunk(c, par, ce, inv_k, sign, wsplat, acc):
        psplat = _splat(par)
        ebase0 = c * ce

        def gbody(g, acc2):
            ridx = _splat(g * 16) + iota16
            e_f = (_splat(g * 16) + iota16).astype(jnp.float32) + (
                ebase0.astype(jnp.float32))
            bvec = ((e_f + 0.5) * inv_k).astype(jnp.int32)
            dvec0 = jnp.bitwise_and(iota16, 15)
            dot = (plsc.load_gather(rows, [psplat, ridx, dvec0])
                   * plsc.load_gather(qbuf, [bvec, dvec0]))
            for d in range(1, 16):
                dvec = jnp.bitwise_and(iota16 + d, 15)
                dot = dot + (plsc.load_gather(rows, [psplat, ridx, dvec])
                             * plsc.load_gather(qbuf, [bvec, dvec]))
            x = dot if sign > 0 else -dot
            return acc2 + wsplat * _log_sigmoid(x)

        return lax.fori_loop(0, ce // 16, gbody, acc)

    def issue_to(tab, off, ce, par):
        pltpu.async_copy(
            tab.at[idxflat.at[pl.ds(off, ce)]],
            rows.at[par, pl.ds(0, ce)], semr.at[par])

    def run_segment(g0, gchunks, tab, ebase, n, ce, inv_k, sign, wslot, acc):
        wsplat = plsc.load_gather(wbuf, [_splat(wslot)])

        def drain(par):
            pltpu.make_async_copy(
                node_t.at[0, pl.ds(0, ce)],
                rows.at[par, pl.ds(0, ce)], semr.at[par]).wait()

        if g0 == 0:
            for p in range(3):
                tb, toff, tce = gchunks[p]
                issue_to(tb, toff, tce, p)

        def cbody(c, acc2):
            par = lax.rem(g0 + c, 4)

            @pl.when(c + 3 < n)
            def _():
                issue_to(tab, ebase + (c + 3) * ce, ce, lax.rem(g0 + c + 3, 4))

            for k in range(min(3, n)):
                tgt = g0 + n + 2 - k
                if tgt < len(gchunks):
                    tb, toff, tce = gchunks[tgt]

                    @pl.when(c == n - 1 - k)
                    def _(tb=tb, toff=toff, tce=tce, tgt=tgt):
                        issue_to(tb, toff, tce, tgt % 4)

            drain(par)
            return compute_chunk(c, par, ce, inv_k, sign, wsplat, acc2)

        return lax.fori_loop(0, n, cbody, acc)

    def block(i):
        j = 1 - i
        wl = wid - i * WPV
        b0 = wl * PPW
        regions = [
            (_REG_NODES, 1, nodes_ib.at[i, pl.ds(b0, PPW)]),
            (_REG_NEIGH, 5, neighs_ib.at[i, pl.ds(b0 * NH, PPW * NH)]),
            (_REG_ROLE0, 3, roles_ib.at[i, 0, pl.ds(b0 * NR, PPW * NR)]),
            (_REG_ROLE1, 3, roles_ib.at[i, 1, pl.ds(b0 * NR, PPW * NR)]),
            (_REG_NEGM, 50, negm.at[i, pl.ds(b0 * NH * NG, PPW * NH * NG)]),
            (_REG_NEG2, 10, neg2.at[i, j, pl.ds(b0 * NG, PPW * NG)]),
            (_REG_NEG3, 50, neg3.at[i, j, pl.ds(b0 * NH * NG, PPW * NH * NG)]),
            (_REG_NEG4A, 30, neg4.at[i, 0, pl.ds(b0 * NR * NG, PPW * NR * NG)]),
            (_REG_NEG4B, 30, neg4.at[i, 1, pl.ds(b0 * NR * NG, PPW * NR * NG)]),
        ]
        for base, n, src in regions:
            pltpu.async_copy(src, idxflat.at[pl.ds(base * CE, n * CE)], sems)
        for base, n, src in regions:
            pltpu.make_async_copy(
                negm.at[0, pl.ds(0, n * CE)],
                idxflat.at[pl.ds(base * CE, n * CE)], sems).wait()
        for q in range(2):
            pltpu.async_copy(
                node_t.at[i].at[idxflat.at[pl.ds(_REG_NODES * CE + q * 128, 128)]],
                qbuf.at[pl.ds(q * 128, 128)], semq)
        for q in range(2):
            pltpu.make_async_copy(node_t.at[0, pl.ds(0, 128)],
                                  qbuf.at[pl.ds(q * 128, 128)], semq).wait()

        inv5 = jnp.float32(1.0 / NH)
        inv50 = jnp.float32(1.0 / (NH * NG))
        inv10 = jnp.float32(1.0 / NG)
        inv3 = jnp.float32(1.0 / NR)
        inv30 = jnp.float32(1.0 / (NR * NG))
        one = jnp.float32(1.0)
        segs = [
            (neigh_t.at[i], _REG_NEIGH, 5, 256, inv5, 1, 0),
            (neigh_t.at[i], _REG_NEGM, 25, 512, inv50, -1, 1),
            (node_t.at[j], _REG_NODES, 1, 256, one, 1, 2),
            (node_t.at[j], _REG_NEG2, 5, 512, inv10, -1, 2),
            (neigh_t.at[j], _REG_NEIGH, 5, 256, inv5, 1, 3),
            (neigh_t.at[j], _REG_NEG3, 25, 512, inv50, -1, 4),
            (node_t.at[0], _REG_ROLE0, 3, 256, inv3, 1, 5),
            (node_t.at[1], _REG_ROLE1, 3, 256, inv3, 1, 5),
            (node_t.at[0], _REG_NEG4A, 15, 512, inv30, -1, 6),
            (node_t.at[1], _REG_NEG4B, 15, 512, inv30, -1, 6),
        ]
        gchunks = []
        g0s = []
        for tab, reg0, n, ce, inv_k, sign, wslot in segs:
            g0s.append(len(gchunks))
            for c in range(n):
                gchunks.append((tab, reg0 * CE + c * ce, ce))
        acc = jnp.zeros((16,), jnp.float32)
        for g0, (tab, reg0, n, ce, inv_k, sign, wslot) in zip(g0s, segs):
            acc = run_segment(g0, gchunks, tab, reg0 * CE, n, ce,
                              inv_k, sign, wslot, acc)
        accv[...] = acc

    @pl.when(wid < WPV)
    def _():
        block(0)

    @pl.when(wid >= WPV)
    def _():
        block(1)

    pltpu.sync_copy(accv, out_hbm.at[wid])


def kernel(node_emb_tables, neigh_emb_tables, hyp1, hyp2, hyp3, count,
           shuffle_indices_nets, nodes_idx_nets, neigh_idx_nets,
           node_role_nets, neg_main, neg2, neg3, neg4):
    bidx = [lax.dynamic_slice_in_dim(shuffle_indices_nets[i], count, B)
            for i in range(NV)]
    nodes_ib = jnp.stack([nodes_idx_nets[i][bidx[i]] for i in range(NV)])
    neighs_ib = jnp.stack(
        [neigh_idx_nets[i][bidx[i]].reshape(-1) for i in range(NV)])
    roles_ib = jnp.stack([
        jnp.stack([node_role_nets[i, jj][bidx[i]].reshape(-1)
                   for jj in range(NV)]) for i in range(NV)])

    Bf = jnp.float32(B)
    scale = jnp.float32(-0.1)
    wvec = jnp.stack([
        1.0 / (NH * Bf), 1.0 / Bf, hyp1 / Bf, hyp2 / (NH * Bf), hyp2 / Bf,
        hyp3 / (NR * Bf), hyp3 / Bf,
        0.0, 0.0, 0.0, 0.0, 0.0, 0.0, 0.0, 0.0, 0.0,
    ]).astype(jnp.float32) * scale

    out = _sc_loss(node_emb_tables, neigh_emb_tables,
                   nodes_ib.astype(jnp.int32), neighs_ib.astype(jnp.int32),
                   roles_ib.astype(jnp.int32), neg_main, neg2, neg3, neg4,
                   wvec)
    return jnp.sum(out)


# speedup vs baseline: 1.0671x; 1.0011x over previous
import functools

import jax
import jax.numpy as jnp
from jax import lax
from jax.experimental import pallas as pl
from jax.experimental.pallas import tpu as pltpu
from jax.experimental.pallas import tpu_sc as plsc

NV, D, NN, B = 2, 16, 1000000, 4096
NH, NR, NG = 5, 3, 10
NW = 32
WPV = NW // NV
PPW = B // WPV
CE = 256
_REG_NODES, _REG_NEIGH, _REG_ROLE0, _REG_ROLE1 = 0, 1, 6, 9
_REG_NEGM, _REG_NEG2, _REG_NEG3, _REG_NEG4A, _REG_NEG4B = 12, 62, 72, 122, 152
N_CHUNKS = 182
IDXW = N_CHUNKS * CE

_mesh = plsc.VectorSubcoreMesh(
    core_axis_name="c", subcore_axis_name="s", num_cores=2, num_subcores=16)


def _splat(v):
    return jnp.full((16,), v, jnp.int32)


def _log_sigmoid(x):
    u = jnp.exp(-jnp.abs(x))
    t = 1.0 + u
    bits = lax.bitcast_convert_type(t, jnp.int32)
    ex = ((bits >> 23) - 127).astype(jnp.float32)
    m = lax.bitcast_convert_type((bits & 0x007FFFFF) | 0x3F800000, jnp.float32)
    s = (m - 1.0) / (m + 1.0)
    s2 = s * s
    p = jnp.float32(1.0 / 9.0)
    p = jnp.float32(1.0 / 7.0) + s2 * p
    p = jnp.float32(1.0 / 5.0) + s2 * p
    p = jnp.float32(1.0 / 3.0) + s2 * p
    p = jnp.float32(1.0) + s2 * p
    l1p = ex * jnp.float32(0.6931471805599453) + 2.0 * s * p
    return jnp.minimum(x, 0.0) - l1p


@functools.partial(
    pl.kernel,
    out_type=jax.ShapeDtypeStruct((NW, 16), jnp.float32),
    mesh=_mesh,
    compiler_params=pltpu.CompilerParams(
        use_tc_tiling_on_sc=False, needs_layout_passes=False),
    scratch_types=[
        pltpu.VMEM((IDXW,), jnp.int32),
        pltpu.VMEM((PPW, 16), jnp.float32),
        pltpu.VMEM((4, 2 * CE, 16), jnp.float32),
        pltpu.VMEM((16,), jnp.float32),
        pltpu.VMEM((16,), jnp.float32),
        pltpu.SemaphoreType.DMA,
        pltpu.SemaphoreType.DMA,
        pltpu.SemaphoreType.DMA((4,)),
    ],
)
def _sc_loss(node_t, neigh_t, nodes_ib, neighs_ib, roles_ib,
             negm, neg2, neg3, neg4, wvec_hbm, out_hbm,
             idxflat, qbuf, rows, wbuf, accv, sems, semq, semr):
    wid = lax.axis_index("s") * 2 + lax.axis_index("c")
    pltpu.sync_copy(wvec_hbm, wbuf)
    iota16 = lax.iota(jnp.int32, 16)

    def compute_chunk(c, par, ce, inv_k, sign, wsplat, acc):
        psplat = _splat(par)
        ebase0 = c * ce

        def gbody(g, acc2):
            ridx = _splat(g * 16) + iota16
            e_f = (_splat(g * 16) + iota16).astype(jnp.float32) + (
                ebase0.astype(jnp.float32))
            bvec = ((e_f + 0.5) * inv_k).astype(jnp.int32)
            dvec0 = jnp.bitwise_and(iota16, 15)
            dot = (plsc.load_gather(rows, [psplat, ridx, dvec0])
                   * plsc.load_gather(qbuf, [bvec, dvec0]))
            for d in range(1, 16):
                dvec = jnp.bitwise_and(iota16 + d, 15)
                dot = dot + (plsc.load_gather(rows, [psplat, ridx, dvec])
                             * plsc.load_gather(qbuf, [bvec, dvec]))
            x = dot if sign > 0 else -dot
            return acc2 + wsplat * _log_sigmoid(x)

        return lax.fori_loop(0, ce // 16, gbody, acc)

    def issue_to(tab, off, ce, par):
        pltpu.async_copy(
            tab.at[idxflat.at[pl.ds(off, ce)]],
            rows.at[par, pl.ds(0, ce)], semr.at[par])

    def run_segment(g0, gchunks, tab, ebase, n, ce, inv_k, sign, wslot, acc):
        wsplat = plsc.load_gather(wbuf, [_splat(wslot)])

        def drain(par):
            pltpu.make_async_copy(
                node_t.at[0, pl.ds(0, ce)],
                rows.at[par, pl.ds(0, ce)], semr.at[par]).wait()

        def cbody(c, acc2):
            par = lax.rem(g0 + c, 4)

            @pl.when(c + 3 < n)
            def _():
                issue_to(tab, ebase + (c + 3) * ce, ce, lax.rem(g0 + c + 3, 4))

            for k in range(min(3, n)):
                tgt = g0 + n + 2 - k
                if tgt < len(gchunks):
                    tb, toff, tce = gchunks[tgt]

                    @pl.when(c == n - 1 - k)
                    def _(tb=tb, toff=toff, tce=tce, tgt=tgt):
                        issue_to(tb, toff, tce, tgt % 4)

            drain(par)
            return compute_chunk(c, par, ce, inv_k, sign, wsplat, acc2)

        return lax.fori_loop(0, n, cbody, acc)

    def block(i):
        j = 1 - i
        wl = wid - i * WPV
        b0 = wl * PPW
        regions = [
            (_REG_NODES, 1, nodes_ib.at[i, pl.ds(b0, PPW)]),
            (_REG_NEIGH, 5, neighs_ib.at[i, pl.ds(b0 * NH, PPW * NH)]),
            (_REG_ROLE0, 3, roles_ib.at[i, 0, pl.ds(b0 * NR, PPW * NR)]),
            (_REG_ROLE1, 3, roles_ib.at[i, 1, pl.ds(b0 * NR, PPW * NR)]),
            (_REG_NEGM, 50, negm.at[i, pl.ds(b0 * NH * NG, PPW * NH * NG)]),
            (_REG_NEG2, 10, neg2.at[i, j, pl.ds(b0 * NG, PPW * NG)]),
            (_REG_NEG3, 50, neg3.at[i, j, pl.ds(b0 * NH * NG, PPW * NH * NG)]),
            (_REG_NEG4A, 30, neg4.at[i, 0, pl.ds(b0 * NR * NG, PPW * NR * NG)]),
            (_REG_NEG4B, 30, neg4.at[i, 1, pl.ds(b0 * NR * NG, PPW * NR * NG)]),
        ]
        for base, n, src in regions:
            pltpu.async_copy(src, idxflat.at[pl.ds(base * CE, n * CE)], sems)
        for base, n, src in regions:
            pltpu.make_async_copy(
                negm.at[0, pl.ds(0, n * CE)],
                idxflat.at[pl.ds(base * CE, n * CE)], sems).wait()
        for q in range(2):
            pltpu.async_copy(
                node_t.at[i].at[idxflat.at[pl.ds(_REG_NODES * CE + q * 128, 128)]],
                qbuf.at[pl.ds(q * 128, 128)], semq)

        inv5 = jnp.float32(1.0 / NH)
        inv50 = jnp.float32(1.0 / (NH * NG))
        inv10 = jnp.float32(1.0 / NG)
        inv3 = jnp.float32(1.0 / NR)
        inv30 = jnp.float32(1.0 / (NR * NG))
        one = jnp.float32(1.0)
        segs = [
            (neigh_t.at[i], _REG_NEIGH, 5, 256, inv5, 1, 0),
            (neigh_t.at[i], _REG_NEGM, 25, 512, inv50, -1, 1),
            (node_t.at[j], _REG_NODES, 1, 256, one, 1, 2),
            (node_t.at[j], _REG_NEG2, 5, 512, inv10, -1, 2),
            (neigh_t.at[j], _REG_NEIGH, 5, 256, inv5, 1, 3),
            (neigh_t.at[j], _REG_NEG3, 25, 512, inv50, -1, 4),
            (node_t.at[0], _REG_ROLE0, 3, 256, inv3, 1, 5),
            (node_t.at[1], _REG_ROLE1, 3, 256, inv3, 1, 5),
            (node_t.at[0], _REG_NEG4A, 15, 512, inv30, -1, 6),
            (node_t.at[1], _REG_NEG4B, 15, 512, inv30, -1, 6),
        ]
        gchunks = []
        g0s = []
        for tab, reg0, n, ce, inv_k, sign, wslot in segs:
            g0s.append(len(gchunks))
            for c in range(n):
                gchunks.append((tab, reg0 * CE + c * ce, ce))
        for p in range(3):
            tb, toff, tce = gchunks[p]
            issue_to(tb, toff, tce, p)
        for q in range(2):
            pltpu.make_async_copy(node_t.at[0, pl.ds(0, 128)],
                                  qbuf.at[pl.ds(q * 128, 128)], semq).wait()
        acc = jnp.zeros((16,), jnp.float32)
        for g0, (tab, reg0, n, ce, inv_k, sign, wslot) in zip(g0s, segs):
            acc = run_segment(g0, gchunks, tab, reg0 * CE, n, ce,
                              inv_k, sign, wslot, acc)
        accv[...] = acc

    @pl.when(wid < WPV)
    def _():
        block(0)

    @pl.when(wid >= WPV)
    def _():
        block(1)

    pltpu.sync_copy(accv, out_hbm.at[wid])


def kernel(node_emb_tables, neigh_emb_tables, hyp1, hyp2, hyp3, count,
           shuffle_indices_nets, nodes_idx_nets, neigh_idx_nets,
           node_role_nets, neg_main, neg2, neg3, neg4):
    bidx = [lax.dynamic_slice_in_dim(shuffle_indices_nets[i], count, B)
            for i in range(NV)]
    nodes_ib = jnp.stack([nodes_idx_nets[i][bidx[i]] for i in range(NV)])
    neighs_ib = jnp.stack(
        [neigh_idx_nets[i][bidx[i]].reshape(-1) for i in range(NV)])
    roles_ib = jnp.stack([
        jnp.stack([node_role_nets[i, jj][bidx[i]].reshape(-1)
                   for jj in range(NV)]) for i in range(NV)])

    Bf = jnp.float32(B)
    scale = jnp.float32(-0.1)
    wvec = jnp.stack([
        1.0 / (NH * Bf), 1.0 / Bf, hyp1 / Bf, hyp2 / (NH * Bf), hyp2 / Bf,
        hyp3 / (NR * Bf), hyp3 / Bf,
        0.0, 0.0, 0.0, 0.0, 0.0, 0.0, 0.0, 0.0, 0.0,
    ]).astype(jnp.float32) * scale

    out = _sc_loss(node_emb_tables, neigh_emb_tables,
                   nodes_ib.astype(jnp.int32), neighs_ib.astype(jnp.int32),
                   roles_ib.astype(jnp.int32), neg_main, neg2, neg3, neg4,
                   wvec)
    return jnp.sum(out)
